# baseline (device time: 90300 ns/iter reference)
import jax
import jax.numpy as jnp
from jax import lax
from jax.experimental import pallas as pl
from jax.experimental.pallas import tpu as pltpu

N_DEV = 4
BLOCK_M = 512
EPS = 1e-5

_DEVID = getattr(pl, "DeviceIdType", None) or pltpu.DeviceIdType
_sem_signal = getattr(pl, "semaphore_signal", None) or pltpu.semaphore_signal
_sem_wait = getattr(pl, "semaphore_wait", None) or pltpu.semaphore_wait


def kernel(x, gamma, beta):
    m, n_loc = x.shape
    n_glob = float(N_DEV * n_loc)
    nblk = m // BLOCK_M
    g2 = gamma.reshape(1, n_loc)
    b2 = beta.reshape(1, n_loc)

    def body(x_ref, g_ref, b_ref, out_ref,
             loc_ref, snd_ref, comm_ref, ms_ref, send_sems, recv_sems):
        p = pl.program_id(0)
        blk = pl.program_id(1)
        my = lax.axis_index("i")

        @pl.when(p == 0)
        def _pass0():
            xs = x_ref[...]
            rows = pl.ds(blk * BLOCK_M, BLOCK_M)
            loc_ref[rows, 0:1] = jnp.sum(xs, axis=1, keepdims=True)
            loc_ref[rows, 1:2] = jnp.sum(xs * xs, axis=1, keepdims=True)

        @pl.when((p == 0) & (blk == nblk - 1))
        def _exchange():
            snd_ref[...] = jnp.transpose(loc_ref[...], (1, 0))

            barrier = pltpu.get_barrier_semaphore()
            for k in range(1, N_DEV):
                _sem_signal(
                    barrier, inc=1,
                    device_id=((my + k) % N_DEV,),
                    device_id_type=_DEVID.MESH,
                )
            _sem_wait(barrier, N_DEV - 1)

            sends = []
            for k in range(1, N_DEV):
                tgt = (my + k) % N_DEV
                rdma = pltpu.make_async_remote_copy(
                    src_ref=snd_ref,
                    dst_ref=comm_ref.at[my],
                    send_sem=send_sems.at[k - 1],
                    recv_sem=recv_sems.at[my],
                    device_id=(tgt,),
                    device_id_type=_DEVID.MESH,
                )
                rdma.start()
                sends.append(rdma)
            for rdma in sends:
                rdma.wait_send()

            tot = snd_ref[...]
            for k in range(1, N_DEV):
                src = (my + k) % N_DEV
                recv = pltpu.make_async_remote_copy(
                    src_ref=snd_ref,
                    dst_ref=comm_ref.at[src],
                    send_sem=send_sems.at[0],
                    recv_sem=recv_sems.at[src],
                    device_id=(src,),
                    device_id_type=_DEVID.MESH,
                )
                recv.wait_recv()
                tot = tot + comm_ref[src]

            mean = tot[0:1, :] / n_glob
            ex2 = tot[1:2, :] / n_glob
            rstd = lax.rsqrt(ex2 - mean * mean + EPS)
            ms_ref[...] = jnp.transpose(
                jnp.concatenate([mean, rstd], axis=0), (1, 0)
            )

        @pl.when(p == 1)
        def _pass1():
            xs = x_ref[...]
            rows = pl.ds(blk * BLOCK_M, BLOCK_M)
            mu = ms_ref[rows, 0:1]
            rs = ms_ref[rows, 1:2]
            out_ref[...] = (
                g_ref[...] * ((xs - mu) * rs) + b_ref[...]
            ).astype(out_ref.dtype)

    return pl.pallas_call(
        body,
        grid=(2, nblk),
        out_shape=jax.ShapeDtypeStruct((m, n_loc), jnp.bfloat16),
        in_specs=[
            pl.BlockSpec((BLOCK_M, n_loc), lambda p, b: (b, 0)),
            pl.BlockSpec((1, n_loc), lambda p, b: (0, 0)),
            pl.BlockSpec((1, n_loc), lambda p, b: (0, 0)),
        ],
        out_specs=pl.BlockSpec(
            (BLOCK_M, n_loc), lambda p, b: (jnp.where(p == 0, 0, b), 0)
        ),
        scratch_shapes=[
            pltpu.VMEM((m, 2), jnp.float32),
            pltpu.VMEM((2, m), jnp.float32),
            pltpu.VMEM((N_DEV, 2, m), jnp.float32),
            pltpu.VMEM((m, 2), jnp.float32),
            pltpu.SemaphoreType.DMA((N_DEV - 1,)),
            pltpu.SemaphoreType.DMA((N_DEV,)),
        ],
        compiler_params=pltpu.CompilerParams(collective_id=0),
    )(x, g2, b2)


# device time: 73974 ns/iter; 1.2207x vs baseline; 1.2207x over previous
import jax
import jax.numpy as jnp
from jax import lax
from jax.experimental import pallas as pl
from jax.experimental.pallas import tpu as pltpu

N_DEV = 4
BLOCK_M = 512
EPS = 1e-5

_DEVID = getattr(pl, "DeviceIdType", None) or pltpu.DeviceIdType
_sem_signal = getattr(pl, "semaphore_signal", None) or pltpu.semaphore_signal
_sem_wait = getattr(pl, "semaphore_wait", None) or pltpu.semaphore_wait


def kernel(x, gamma, beta):
    m, n_loc = x.shape
    n_glob = float(N_DEV * n_loc)
    nblk = m // BLOCK_M
    g2 = gamma.reshape(1, n_loc)
    b2 = beta.reshape(1, n_loc)

    def body(x_ref, g_ref, b_ref, out_ref,
             loc_ref, snd_ref, comm_ref, ms_ref, xc_ref,
             send_sems, recv_sems):
        p = pl.program_id(0)
        blk = pl.program_id(1)
        my = lax.axis_index("i")

        @pl.when(p == 0)
        def _pass0():
            xs = x_ref[...]
            rows = pl.ds(blk * BLOCK_M, BLOCK_M)
            loc_ref[rows, 0:1] = jnp.sum(xs, axis=1, keepdims=True)
            loc_ref[rows, 1:2] = jnp.sum(xs * xs, axis=1, keepdims=True)
            xc_ref[rows, :] = xs.astype(jnp.bfloat16)

        @pl.when((p == 0) & (blk == nblk - 1))
        def _exchange():
            snd_ref[...] = jnp.transpose(loc_ref[...], (1, 0))

            barrier = pltpu.get_barrier_semaphore()
            for k in range(1, N_DEV):
                _sem_signal(
                    barrier, inc=1,
                    device_id=((my + k) % N_DEV,),
                    device_id_type=_DEVID.MESH,
                )
            _sem_wait(barrier, N_DEV - 1)

            sends = []
            for k in range(1, N_DEV):
                tgt = (my + k) % N_DEV
                rdma = pltpu.make_async_remote_copy(
                    src_ref=snd_ref,
                    dst_ref=comm_ref.at[my],
                    send_sem=send_sems.at[k - 1],
                    recv_sem=recv_sems.at[my],
                    device_id=(tgt,),
                    device_id_type=_DEVID.MESH,
                )
                rdma.start()
                sends.append(rdma)
            for rdma in sends:
                rdma.wait_send()

            tot = snd_ref[...]
            for k in range(1, N_DEV):
                src = (my + k) % N_DEV
                recv = pltpu.make_async_remote_copy(
                    src_ref=snd_ref,
                    dst_ref=comm_ref.at[src],
                    send_sem=send_sems.at[0],
                    recv_sem=recv_sems.at[src],
                    device_id=(src,),
                    device_id_type=_DEVID.MESH,
                )
                recv.wait_recv()
                tot = tot + comm_ref[src]

            mean = tot[0:1, :] / n_glob
            ex2 = tot[1:2, :] / n_glob
            rstd = lax.rsqrt(ex2 - mean * mean + EPS)
            ms_ref[...] = jnp.transpose(
                jnp.concatenate([mean, rstd], axis=0), (1, 0)
            )

        @pl.when(p == 1)
        def _pass1():
            rows = pl.ds(blk * BLOCK_M, BLOCK_M)
            xs = xc_ref[rows, :].astype(jnp.float32)
            mu = ms_ref[rows, 0:1]
            rs = ms_ref[rows, 1:2]
            out_ref[...] = (
                g_ref[...] * ((xs - mu) * rs) + b_ref[...]
            ).astype(out_ref.dtype)

    return pl.pallas_call(
        body,
        grid=(2, nblk),
        out_shape=jax.ShapeDtypeStruct((m, n_loc), jnp.bfloat16),
        in_specs=[
            pl.BlockSpec((BLOCK_M, n_loc),
                         lambda p, b: (jnp.where(p == 0, b, nblk - 1), 0)),
            pl.BlockSpec((1, n_loc), lambda p, b: (0, 0)),
            pl.BlockSpec((1, n_loc), lambda p, b: (0, 0)),
        ],
        out_specs=pl.BlockSpec(
            (BLOCK_M, n_loc), lambda p, b: (jnp.where(p == 0, 0, b), 0)
        ),
        scratch_shapes=[
            pltpu.VMEM((m, 2), jnp.float32),
            pltpu.VMEM((2, m), jnp.float32),
            pltpu.VMEM((N_DEV, 2, m), jnp.float32),
            pltpu.VMEM((m, 2), jnp.float32),
            pltpu.VMEM((m, n_loc), jnp.bfloat16),
            pltpu.SemaphoreType.DMA((N_DEV - 1,)),
            pltpu.SemaphoreType.DMA((N_DEV,)),
        ],
        compiler_params=pltpu.CompilerParams(
            collective_id=0,
            vmem_limit_bytes=60 * 1024 * 1024,
        ),
    )(x, g2, b2)
